# baseline (speedup 1.0000x reference)
import functools

import jax
import jax.numpy as jnp
from jax import lax
from jax.experimental import pallas as pl
from jax.experimental.pallas import tpu as pltpu


def _round_up(x, m):
    return (x + m - 1) // m * m


def _fused_kernel(x_ref, w1_ref, b1_ref, w2_ref, b2_ref, o_ref, *, n_classes):
    xb = x_ref[...].astype(jnp.bfloat16)

    h = lax.dot_general(
        xb, w1_ref[...],
        dimension_numbers=(((1,), (1,)), ((), ())),
        preferred_element_type=jnp.float32,
    )
    h = jnp.maximum(h + b1_ref[...], 0.0).astype(jnp.bfloat16)

    logits = lax.dot_general(
        h, w2_ref[...],
        dimension_numbers=(((1,), (1,)), ((), ())),
        preferred_element_type=jnp.float32,
    ) + b2_ref[...]

    m = jnp.max(logits, axis=1, keepdims=True)
    e = jnp.exp(logits - m)
    denom = jnp.sum(e, axis=1, keepdims=True)
    o_ref[...] = (e * (1.0 / denom))[:, :n_classes]


def kernel(x, w1, b1, w2, b2):
    B, in_f = x.shape
    hid = w1.shape[0]
    C = w2.shape[0]
    Cp = _round_up(C, 128)

    w1b = w1.astype(jnp.bfloat16)
    w2b = jnp.pad(w2.astype(jnp.bfloat16), ((0, Cp - C), (0, 0)))
    b1r = b1.reshape(1, hid).astype(jnp.float32)
    b2r = jnp.pad(b2.astype(jnp.float32), (0, Cp - C),
                  constant_values=-1e30).reshape(1, Cp)

    tb = min(1024, B)
    grid = (pl.cdiv(B, tb),)

    x_tile = _round_up(tb, 8) * _round_up(in_f, 128) * 4
    o_tile = _round_up(tb, 8) * Cp * 4
    vmem_limit_bytes = int(min(
        max(2 * (x_tile + o_tile) + (6 << 20), 32 << 20), 100 << 20))

    return pl.pallas_call(
        functools.partial(_fused_kernel, n_classes=C),
        out_shape=jax.ShapeDtypeStruct((B, C), jnp.float32),
        grid=grid,
        in_specs=[
            pl.BlockSpec((tb, in_f), lambda i: (i, 0)),
            pl.BlockSpec((hid, in_f), lambda i: (0, 0)),
            pl.BlockSpec((1, hid), lambda i: (0, 0)),
            pl.BlockSpec((Cp, hid), lambda i: (0, 0)),
            pl.BlockSpec((1, Cp), lambda i: (0, 0)),
        ],
        out_specs=pl.BlockSpec((tb, C), lambda i: (i, 0)),
        compiler_params=pltpu.CompilerParams(
            dimension_semantics=("parallel",),
            vmem_limit_bytes=vmem_limit_bytes,
        ),
        cost_estimate=pl.CostEstimate(
            flops=2 * B * (in_f * hid + hid * C),
            transcendentals=B * Cp,
            bytes_accessed=4 * (B * in_f + B * C) + 2 * (hid * in_f + Cp * hid),
        ),
    )(x, w1b, b1r, w2b, b2r)
